# baseline (speedup 1.0000x reference)
import functools

import numpy as np
import jax
import jax.numpy as jnp
from jax import lax
from jax.experimental import pallas as pl
from jax.experimental.pallas import tpu as pltpu
from jax.experimental.pallas import tpu_sc as plsc

_D_MODEL = 1024
_LANES = 16


def _positional_encoding_np(seq_len, d_model):
    pos = np.arange(seq_len, dtype=np.float32)[:, None]
    i = np.arange(d_model // 2, dtype=np.float32)[None, :]
    div = np.exp(-(2.0 * i / d_model) * np.log(10000.0))
    ang = pos * div
    pe = np.zeros((seq_len, d_model), dtype=np.float32)
    pe[:, 0::2] = np.sin(ang)
    pe[:, 1::2] = np.cos(ang)
    return pe


@functools.lru_cache(maxsize=None)
def _build(batch, seq, vocab, d):
    info = plsc.get_sparse_core_info()
    nc, ns = info.num_cores, info.num_subcores
    nw = nc * ns
    pos_per_w = seq // nw
    chunk = 32
    n_sub = pos_per_w // chunk
    n_col = d // _LANES
    scale = float(np.sqrt(d))
    mesh = plsc.VectorSubcoreMesh(core_axis_name="c", subcore_axis_name="s")

    @functools.partial(
        pl.kernel,
        mesh=mesh,
        out_type=jax.ShapeDtypeStruct((batch * seq, d), jnp.float32),
        scratch_types=[
            pltpu.VMEM((chunk,), jnp.int32),
            pltpu.VMEM((chunk, d), jnp.float32),
            pltpu.VMEM((chunk, d), jnp.float32),
            pltpu.SemaphoreType.DMA,
        ],
    )
    def emb_kernel(x_hbm, table_hbm, pe_hbm, out_hbm, idx_v, pe_v, rows_v, sem):
        wid = lax.axis_index("s") * nc + lax.axis_index("c")
        for s in range(n_sub):
            pos0 = wid * pos_per_w + s * chunk
            pltpu.sync_copy(pe_hbm.at[pl.ds(pos0, chunk)], pe_v)
            for b in range(batch):
                row0 = b * seq + pos0
                pltpu.sync_copy(x_hbm.at[pl.ds(row0, chunk)], idx_v)
                pltpu.async_copy(table_hbm.at[idx_v], rows_v, sem).wait()

                def body(r, _):
                    for c in range(n_col):
                        off = c * _LANES
                        rv = rows_v[r, pl.ds(off, _LANES)]
                        pv = pe_v[r, pl.ds(off, _LANES)]
                        rows_v[r, pl.ds(off, _LANES)] = rv * scale + pv
                    return 0

                lax.fori_loop(0, chunk, body, 0)
                pltpu.sync_copy(rows_v, out_hbm.at[pl.ds(row0, chunk)])

    return emb_kernel


def kernel(x, table):
    b, s = x.shape
    v, d = table.shape
    pe = jnp.asarray(_positional_encoding_np(s, d))
    out = _build(b, s, v, d)(x.reshape(b * s), table, pe)
    return out.reshape(b, s, d)


# speedup vs baseline: 1.3026x; 1.3026x over previous
import functools

import numpy as np
import jax
import jax.numpy as jnp
from jax import lax
from jax.experimental import pallas as pl
from jax.experimental.pallas import tpu as pltpu
from jax.experimental.pallas import tpu_sc as plsc

_LANES = 16
_CHUNK = 16
_NBUF = 4


def _positional_encoding_np(seq_len, d_model):
    pos = np.arange(seq_len, dtype=np.float32)[:, None]
    i = np.arange(d_model // 2, dtype=np.float32)[None, :]
    div = np.exp(-(2.0 * i / d_model) * np.log(10000.0))
    ang = pos * div
    pe = np.zeros((seq_len, d_model), dtype=np.float32)
    pe[:, 0::2] = np.sin(ang)
    pe[:, 1::2] = np.cos(ang)
    return pe


@functools.lru_cache(maxsize=None)
def _build(batch, seq, vocab, d):
    info = plsc.get_sparse_core_info()
    nc, ns = info.num_cores, info.num_subcores
    nw = nc * ns
    pos_per_w = seq // nw
    n_sub = pos_per_w // _CHUNK
    n_tasks = n_sub * batch
    n_col = d // _LANES
    scale = float(np.sqrt(d))
    mesh = plsc.VectorSubcoreMesh(core_axis_name="c", subcore_axis_name="s")

    scratch = (
        [pltpu.VMEM((_CHUNK,), jnp.int32) for _ in range(_NBUF)]
        + [pltpu.VMEM((_CHUNK, d), jnp.float32) for _ in range(_NBUF)]
        + [pltpu.VMEM((_CHUNK, d), jnp.float32) for _ in range(2)]
        + [pltpu.SemaphoreType.DMA for _ in range(_NBUF)]
        + [pltpu.SemaphoreType.DMA for _ in range(_NBUF)]
        + [pltpu.SemaphoreType.DMA for _ in range(2)]
    )

    @functools.partial(
        pl.kernel,
        mesh=mesh,
        out_type=jax.ShapeDtypeStruct((batch * seq, d), jnp.float32),
        scratch_types=scratch,
    )
    def emb_kernel(x_hbm, table_hbm, pe_hbm, out_hbm, *refs):
        idx_v = refs[0:_NBUF]
        rows_v = refs[_NBUF:2 * _NBUF]
        pe_v = refs[2 * _NBUF:2 * _NBUF + 2]
        g_sem = refs[2 * _NBUF + 2:3 * _NBUF + 2]
        st_sem = refs[3 * _NBUF + 2:4 * _NBUF + 2]
        pe_sem = refs[4 * _NBUF + 2:]

        wid = lax.axis_index("s") * nc + lax.axis_index("c")
        pos_base = wid * pos_per_w

        def task_row0(t):
            return (t % batch) * seq + pos_base + (t // batch) * _CHUNK

        def start_gather(t):
            p = t % _NBUF
            pltpu.sync_copy(x_hbm.at[pl.ds(task_row0(t), _CHUNK)], idx_v[p])
            return pltpu.async_copy(table_hbm.at[idx_v[p]], rows_v[p], g_sem[p])

        def start_pe(s):
            q = s % 2
            return pltpu.async_copy(
                pe_hbm.at[pl.ds(pos_base + s * _CHUNK, _CHUNK)], pe_v[q], pe_sem[q])

        g_h = {0: start_gather(0), 1: start_gather(1)}
        pe_h = {0: start_pe(0)}
        pe_h[0].wait()
        if n_sub > 1:
            pe_h[1] = start_pe(1)

        st_h = {}
        for t in range(n_tasks):
            p = t % _NBUF
            if t + 2 < n_tasks:
                if t - 2 >= 0:
                    st_h[t - 2].wait()
                g_h[t + 2] = start_gather(t + 2)
            s = t // batch
            q = s % 2
            if t % batch == 0 and s > 0:
                pe_h[s].wait()
                if s + 1 < n_sub:
                    pe_h[s + 1] = start_pe(s + 1)
            g_h[t].wait()

            def body(r, _):
                for c in range(n_col):
                    off = c * _LANES
                    rv = rows_v[p][r, pl.ds(off, _LANES)]
                    pv = pe_v[q][r, pl.ds(off, _LANES)]
                    rows_v[p][r, pl.ds(off, _LANES)] = rv * scale + pv
                return 0

            lax.fori_loop(0, _CHUNK, body, 0)
            st_h[t] = pltpu.async_copy(
                rows_v[p], out_hbm.at[pl.ds(task_row0(t), _CHUNK)], st_sem[p])

        for t in range(max(0, n_tasks - _NBUF), n_tasks):
            st_h[t].wait()

    return emb_kernel


def kernel(x, table):
    b, s = x.shape
    v, d = table.shape
    pe = jnp.asarray(_positional_encoding_np(s, d))
    out = _build(b, s, v, d)(x.reshape(b * s), table, pe)
    return out.reshape(b, s, d)
